# Initial kernel scaffold; baseline (speedup 1.0000x reference)
import jax
import jax.numpy as jnp
from jax.experimental import pallas as pl


def kernel(x, embedding_q, embedding_r):
    raise NotImplementedError("write your pallas kernel here")


# speedup vs baseline: 4.6272x; 4.6272x over previous
import functools

import jax
import jax.numpy as jnp
from jax import lax
from jax.experimental import pallas as pl
from jax.experimental.pallas import tpu as pltpu
from jax.experimental.pallas import tpu_sc as plsc

_QR_RATIO = 64
_EMB_DIM = 64
_LANES = 16
_NC = 2
_NS = 16
_NW = _NC * _NS

_B = 16384 * 26
_PW = _B // _NW
_C = 512
_NCH = _PW // _C
_GSZ = 128
_NG = _C // _GSZ


def _body(x_hbm, embq_hbm, embr_hbm, out_hbm, idx_v, qidx_v, ridx_v,
          rows_q, rows_r, sem):
    wid = lax.axis_index("s") * _NC + lax.axis_index("c")
    base_w = wid * _PW

    def chunk(ch, carry):
        base = base_w + ch * _C
        pltpu.sync_copy(x_hbm.at[pl.ds(base, _C)], idx_v)

        for i in range(_C // _LANES):
            v = idx_v[pl.ds(i * _LANES, _LANES)]
            g = i // (_GSZ // _LANES)
            o = (i % (_GSZ // _LANES)) * _LANES
            qidx_v[g, pl.ds(o, _LANES)] = v >> 6
            ridx_v[g, pl.ds(o, _LANES)] = v & (_QR_RATIO - 1)

        copies = []
        for s in range(_NG):
            dst = pl.ds(s * _GSZ, _GSZ)
            copies.append(pltpu.async_copy(
                embq_hbm.at[qidx_v.at[s]], rows_q.at[dst], sem))
            copies.append(pltpu.async_copy(
                embr_hbm.at[ridx_v.at[s]], rows_r.at[dst], sem))
        for cp in copies:
            cp.wait()

        def add_row(i, c):
            for j in range(_EMB_DIM // _LANES):
                blk = pl.ds(j * _LANES, _LANES)
                plsc.addupdate(rows_q.at[i, blk], rows_r[i, blk])
            return c
        lax.fori_loop(0, _C, add_row, 0)

        pltpu.sync_copy(rows_q, out_hbm.at[pl.ds(base, _C)])
        return carry

    lax.fori_loop(0, _NCH, chunk, 0)


@jax.jit
def _qr_embed(x_flat, embedding_q, embedding_r):
    mesh = plsc.VectorSubcoreMesh(
        core_axis_name="c", subcore_axis_name="s",
        num_cores=_NC, num_subcores=_NS)
    return pl.kernel(
        _body,
        out_type=jax.ShapeDtypeStruct((_B, _EMB_DIM), jnp.float32),
        mesh=mesh,
        scratch_types=[
            pltpu.VMEM((_C,), jnp.int32),
            pltpu.VMEM((_NG, _GSZ), jnp.int32),
            pltpu.VMEM((_NG, _GSZ), jnp.int32),
            pltpu.VMEM((_C, _EMB_DIM), jnp.float32),
            pltpu.VMEM((_C, _EMB_DIM), jnp.float32),
            pltpu.SemaphoreType.DMA,
        ],
        compiler_params=pltpu.CompilerParams(use_tc_tiling_on_sc=False),
    )(x_flat, embedding_q, embedding_r)


def kernel(x, embedding_q, embedding_r):
    b, f = x.shape
    x_flat = x.reshape(-1).astype(jnp.int32)
    out = _qr_embed(x_flat, embedding_q, embedding_r)
    return out.reshape(b, f, _EMB_DIM)
